# baseline (device time: 701410 ns/iter reference)
import jax
import jax.numpy as jnp
from jax import lax
from jax.experimental import pallas as pl
from jax.experimental.pallas import tpu as pltpu

N_DEV = 16
N_RINGS = 4
N_HOPS = N_DEV - 1
N_SLOTS = 3


def kernel(x, w_mat):
    m_total, _k_shard = x.shape
    _, n = w_mat.shape
    m_per = m_total // N_DEV
    q = n // N_RINGS

    def body(x_ref, w_ref, out_ref, comm, send_sems, recv_sems, credit_sems):
        my = lax.axis_index("i")
        left = lax.rem(my - 1 + N_DEV, N_DEV)
        right = lax.rem(my + 1, N_DEV)

        def idx_right(k):
            return lax.rem(my - 2 - k + 2 * N_DEV, N_DEV)

        def idx_left(k):
            return lax.rem(my + 2 + k, N_DEV)

        rings = {
            0: (right, left, idx_right),
            1: (right, left, idx_right),
            2: (left, right, idx_left),
            3: (left, right, idx_left),
        }
        ORDER = (0, 2, 1, 3)

        barrier_sem = pltpu.get_barrier_semaphore()
        for nbr in [left, right]:
            pl.semaphore_signal(
                barrier_sem, inc=1,
                device_id=(nbr,), device_id_type=pl.DeviceIdType.MESH,
            )
        pl.semaphore_wait(barrier_sem, 2)

        def partial(g, idx):
            return jnp.dot(
                x_ref[pl.ds(idx * m_per, m_per), :],
                w_ref[:, g * q:(g + 1) * q],
                preferred_element_type=jnp.float32,
            )

        def make_rdma(g, k):
            return pltpu.make_async_remote_copy(
                src_ref=comm.at[g, k % N_SLOTS],
                dst_ref=comm.at[g, (k + 1) % N_SLOTS],
                send_sem=send_sems.at[g, k % N_SLOTS],
                recv_sem=recv_sems.at[g, (k + 1) % N_SLOTS],
                device_id=(rings[g][0],),
                device_id_type=pl.DeviceIdType.MESH,
            )

        rdmas = {}

        for g in ORDER:
            seed_idx = left if g < 2 else right
            comm[g, 0] = partial(g, seed_idx)
            rdmas[(g, 0)] = make_rdma(g, 0)
            rdmas[(g, 0)].start()

        p = {}
        for g in ORDER:
            p[g] = partial(g, rings[g][2](0))

        for k in range(N_HOPS):
            rs = (k + 1) % N_SLOTS
            for g in ORDER:
                rdmas[(g, k)].wait_recv()
                if k < N_HOPS - 1:
                    comm[g, rs] = comm[g, rs] + p[g]
                    if k >= 1:
                        pl.semaphore_wait(credit_sems.at[g], 1)
                    rdmas[(g, k + 1)] = make_rdma(g, k + 1)
                    rdmas[(g, k + 1)].start()
                else:
                    out_ref[:, g * q:(g + 1) * q] = jnp.maximum(
                        comm[g, rs] + p[g], 0.0
                    )
            for g in ORDER:
                rdmas[(g, k)].wait_send()
                if k < N_HOPS - 2:
                    pl.semaphore_signal(
                        credit_sems.at[g], inc=1,
                        device_id=(rings[g][1],),
                        device_id_type=pl.DeviceIdType.MESH,
                    )
            if k < N_HOPS - 1:
                for g in ORDER:
                    p[g] = partial(g, rings[g][2](k + 1))

    return pl.pallas_call(
        body,
        out_shape=jax.ShapeDtypeStruct((m_per, n), jnp.float32),
        in_specs=[
            pl.BlockSpec(memory_space=pltpu.VMEM),
            pl.BlockSpec(memory_space=pltpu.VMEM),
        ],
        out_specs=pl.BlockSpec(memory_space=pltpu.VMEM),
        scratch_shapes=[
            pltpu.VMEM((N_RINGS, N_SLOTS, m_per, q), jnp.float32),
            pltpu.SemaphoreType.DMA((N_RINGS, N_SLOTS)),
            pltpu.SemaphoreType.DMA((N_RINGS, N_SLOTS)),
            pltpu.SemaphoreType.REGULAR((N_RINGS,)),
        ],
        compiler_params=pltpu.CompilerParams(
            collective_id=0, vmem_limit_bytes=100 * 1024 * 1024,
        ),
    )(x, w_mat)
